# baseline (speedup 1.0000x reference)
import jax
import jax.numpy as jnp
from jax.experimental import pallas as pl
from jax.experimental.pallas import tpu as pltpu
from jax.experimental.pallas import tpu_sc as plsc

EMBED_DIM = 32
WINDOW = 128
KSUB = 8


def _gather_sc(table, indices):
    num_indices = indices.shape[0]
    block = WINDOW * KSUB
    idx2d = indices.reshape(num_indices // WINDOW, WINDOW)
    mesh = plsc.VectorSubcoreMesh(core_axis_name="core", subcore_axis_name="subcore")

    @pl.kernel(
        out_type=jax.ShapeDtypeStruct((num_indices, EMBED_DIM), table.dtype),
        mesh=mesh,
        scratch_types=[pltpu.SemaphoreType.DMA],
        compiler_params=pltpu.CompilerParams(use_tc_tiling_on_sc=False),
    )
    def kern(x_hbm, i_hbm, o_hbm, sem):
        def body(i_vmem, o_vmem):
            copies = [
                pltpu.async_copy(
                    x_hbm.at[i_vmem.at[j]],
                    o_vmem.at[pl.ds(j * WINDOW, WINDOW)],
                    sem,
                )
                for j in range(KSUB)
            ]
            for c in copies:
                c.wait()

        pltpu.emit_pipeline(
            body,
            grid=(num_indices // block,),
            in_specs=[pl.BlockSpec((KSUB, WINDOW), index_map=lambda i: (i, 0))],
            out_specs=[pl.BlockSpec((block, EMBED_DIM), index_map=lambda i: (i, 0))],
            core_axis_name=("core", "subcore"),
            dimension_semantics=(pltpu.PARALLEL,),
        )(i_hbm, o_hbm)

    return kern(table, idx2d)


def kernel(tokens, table):
    batch, hist = tokens.shape
    flat = tokens.reshape(batch * hist).astype(jnp.int32)
    out = _gather_sc(table, flat)
    return out.reshape(batch, hist, EMBED_DIM)
